# baseline (speedup 1.0000x reference)
import functools
import math

import jax
import jax.numpy as jnp
from jax import lax
from jax.experimental import pallas as pl
from jax.experimental.pallas import tpu as pltpu
from jax.experimental.pallas import tpu_sc as plsc

BATCH = 4
SEQ_LEN = 8192
EMBED_DIM = 64
MAX_WAVELENGTH = 10000.0

_NC = 2
_NS = 16
_NW = _NC * _NS
_ROWS = BATCH * SEQ_LEN
_BW = _ROWS // _NW
_CH = 256
_NCHUNK = _BW // _CH
_LANES = EMBED_DIM // 16


def _enc_body(out_ref):
    pos = lax.broadcasted_iota(jnp.int32, (SEQ_LEN, EMBED_DIM), 0).astype(jnp.float32)
    col = lax.broadcasted_iota(jnp.int32, (SEQ_LEN, EMBED_DIM), 1)
    expo = (2 * (col // 2)).astype(jnp.float32) / float(EMBED_DIM)
    ln_base = -math.log(MAX_WAVELENGTH)
    timescales = jnp.exp(expo * ln_base)
    angles = pos * timescales
    odd = (col % 2).astype(jnp.float32)
    out_ref[...] = jnp.sin(angles) * (1.0 - odd) + jnp.cos(angles) * odd


def _position_encoding_tc():
    return pl.pallas_call(
        _enc_body,
        out_shape=jax.ShapeDtypeStruct((SEQ_LEN, EMBED_DIM), jnp.float32),
    )()


_mesh = plsc.VectorSubcoreMesh(core_axis_name="c", subcore_axis_name="s")


@functools.partial(
    pl.kernel,
    out_type=jax.ShapeDtypeStruct((_ROWS, EMBED_DIM), jnp.float32),
    mesh=_mesh,
    scratch_types=[
        pltpu.VMEM((_BW,), jnp.int32),
        pltpu.VMEM((_CH, EMBED_DIM), jnp.float32),
        pltpu.VMEM((_CH, EMBED_DIM), jnp.float32),
        pltpu.SemaphoreType.DMA,
    ],
    compiler_params=pltpu.CompilerParams(use_tc_tiling_on_sc=False),
)
def _gather_add(table_hbm, idx_hbm, enc_hbm, out_hbm, idx_v, rows_v, enc_v, sem):
    wid = lax.axis_index("s") * _NC + lax.axis_index("c")
    base = wid * _BW
    enc_base = base % SEQ_LEN
    pltpu.sync_copy(idx_hbm.at[pl.ds(base, _BW)], idx_v)
    for ci in range(_NCHUNK):
        pltpu.async_copy(
            table_hbm.at[idx_v.at[pl.ds(ci * _CH, _CH)]], rows_v, sem
        ).wait()
        pltpu.sync_copy(enc_hbm.at[pl.ds(enc_base + ci * _CH, _CH)], enc_v)

        def _add_row(r, carry):
            for g in range(_LANES):
                sl = (r, pl.ds(g * 16, 16))
                rows_v[sl] = rows_v[sl] + enc_v[sl]
            return carry

        lax.fori_loop(0, _CH, _add_row, 0)
        pltpu.sync_copy(rows_v, out_hbm.at[pl.ds(base + ci * _CH, _CH)])


def kernel(inputs, table):
    idx = inputs.reshape(-1).astype(jnp.int32)
    enc = _position_encoding_tc()
    out = _gather_add(table, idx, enc)
    return out.reshape(BATCH, SEQ_LEN, EMBED_DIM)
